# baseline (speedup 1.0000x reference)
import jax
import jax.numpy as jnp
from jax.experimental import pallas as pl

_BB = 8


def _onehot_body(x_ref, o_ref):
    x = x_ref[...]
    bb, depth, j = o_ref.shape
    d = jax.lax.broadcasted_iota(jnp.int32, (bb, depth, j), 1)
    o_ref[...] = (d == x[:, None, :]).astype(jnp.float32)


def kernel(X_in, ones):
    B, J = X_in.shape
    depth = ones.shape[0]
    return pl.pallas_call(
        _onehot_body,
        grid=(B // _BB,),
        in_specs=[pl.BlockSpec((_BB, J), lambda i: (i, 0))],
        out_specs=pl.BlockSpec((_BB, depth, J), lambda i: (i, 0, 0)),
        out_shape=jax.ShapeDtypeStruct((B, depth, J), jnp.float32),
    )(X_in)
